# baseline (speedup 1.0000x reference)
import jax
import jax.numpy as jnp
from jax.experimental import pallas as pl
from jax.experimental.pallas import tpu as pltpu

D = 1024
F_EXP = 2048
F_SH = 4096
E = 16
N = 32
FBLK = 1024
HF = FBLK // 2
CPE = F_EXP // FBLK
UNITS = E + F_SH // F_EXP


def _moe_kernel(x_ref, wg_ref, w1a_ref, w1b_ref, w2a_ref, w2b_ref,
                wpa_ref, wpb_ref, s1a_ref, s1b_ref, s2a_ref, s2b_ref,
                spa_ref, spb_ref, scores_ref, y_ref, w_scr):
    u = pl.program_id(0)
    f = pl.program_id(1)
    first = jnp.logical_and(u == 0, f == 0)

    @pl.when(first)
    def _gating():
        xf = x_ref[:]
        scores = jnp.dot(xf, wg_ref[:], preferred_element_type=jnp.float32)
        scores_ref[:] = scores[:, :E]
        cols = jax.lax.broadcasted_iota(jnp.int32, (N, 128), 1)
        valid = cols < E
        s_masked = jnp.where(valid, scores, -jnp.inf)
        m = jnp.max(s_masked, axis=1, keepdims=True)
        ex = jnp.where(valid, jnp.exp(s_masked - m), 0.0)
        probs = ex / jnp.sum(ex, axis=1, keepdims=True)
        p1 = jnp.max(jnp.where(valid, probs, -jnp.inf), axis=1, keepdims=True)
        i1 = jnp.min(jnp.where(probs == p1, cols, 128), axis=1, keepdims=True)
        probs2 = jnp.where(cols == i1, -jnp.inf,
                           jnp.where(valid, probs, -jnp.inf))
        p2 = jnp.max(probs2, axis=1, keepdims=True)
        i2 = jnp.min(jnp.where(probs2 == p2, cols, 128), axis=1, keepdims=True)
        sel = jnp.logical_or(cols == i1, cols == i2)
        w = jnp.where(sel, probs, 0.0)
        w = jnp.where(jnp.logical_and(cols >= E, cols < UNITS), 1.0, w)
        w_scr[:] = w
        y_ref[:] = jnp.zeros_like(y_ref)

    xf = x_ref[:]
    cols = jax.lax.broadcasted_iota(jnp.int32, (N, 128), 1)
    wu = jnp.sum(jnp.where(cols == u, w_scr[:], 0.0), axis=1, keepdims=True)

    xf16 = xf.astype(jnp.bfloat16)

    def ffn(a, b, c):
        xh1 = jnp.dot(xf16, a.astype(jnp.bfloat16),
                      preferred_element_type=jnp.float32)
        xh2 = jnp.dot(xf16, b.astype(jnp.bfloat16),
                      preferred_element_type=jnp.float32)
        h = (xh1 * jax.nn.sigmoid(xh1)) * xh2
        y_ref[:] += jnp.dot(h.astype(jnp.bfloat16), c.astype(jnp.bfloat16),
                            preferred_element_type=jnp.float32) * wu

    @pl.when(u < E)
    def _expert():
        ffn(w1a_ref[0], w2a_ref[0], wpa_ref[0])
        ffn(w1b_ref[0], w2b_ref[0], wpb_ref[0])

    @pl.when(u >= E)
    def _shared():
        ffn(s1a_ref[:], s2a_ref[:], spa_ref[:])
        ffn(s1b_ref[:], s2b_ref[:], spb_ref[:])


@jax.jit
def _run(xf, Wg_pad, W1, W2, Wp, S1, S2, Sp):
    def w_map(half):
        def m(u, f):
            e = jnp.minimum(u, E - 1)
            fc = jnp.where(u < E, f, CPE - 1)
            return (e, 0, 2 * fc + half)
        return m

    def wp_map(half):
        def m(u, f):
            e = jnp.minimum(u, E - 1)
            fc = jnp.where(u < E, f, CPE - 1)
            return (e, 2 * fc + half, 0)
        return m

    def s_map(half):
        def m(u, f):
            j = jnp.where(u < E, 0, (u - E) * CPE + f)
            return (0, 2 * j + half)
        return m

    def sp_map(half):
        def m(u, f):
            j = jnp.where(u < E, 0, (u - E) * CPE + f)
            return (2 * j + half, 0)
        return m

    scores, y = pl.pallas_call(
        _moe_kernel,
        grid=(UNITS, CPE),
        in_specs=[
            pl.BlockSpec((N, D), lambda u, f: (0, 0)),
            pl.BlockSpec((D, 128), lambda u, f: (0, 0)),
            pl.BlockSpec((1, D, HF), w_map(0)),
            pl.BlockSpec((1, D, HF), w_map(1)),
            pl.BlockSpec((1, D, HF), w_map(0)),
            pl.BlockSpec((1, D, HF), w_map(1)),
            pl.BlockSpec((1, HF, D), wp_map(0)),
            pl.BlockSpec((1, HF, D), wp_map(1)),
            pl.BlockSpec((D, HF), s_map(0)),
            pl.BlockSpec((D, HF), s_map(1)),
            pl.BlockSpec((D, HF), s_map(0)),
            pl.BlockSpec((D, HF), s_map(1)),
            pl.BlockSpec((HF, D), sp_map(0)),
            pl.BlockSpec((HF, D), sp_map(1)),
        ],
        out_specs=[
            pl.BlockSpec((N, E), lambda u, f: (0, 0)),
            pl.BlockSpec((N, D), lambda u, f: (0, 0)),
        ],
        out_shape=[
            jax.ShapeDtypeStruct((N, E), jnp.float32),
            jax.ShapeDtypeStruct((N, D), jnp.float32),
        ],
        scratch_shapes=[pltpu.VMEM((N, 128), jnp.float32)],
        compiler_params=pltpu.CompilerParams(
            dimension_semantics=("arbitrary", "arbitrary"),
        ),
    )(xf, Wg_pad, W1, W1, W2, W2, Wp, Wp, S1, S1, S2, S2, Sp, Sp)
    return scores, y


def kernel(x, Wg, W1, W2, Wp, S1, S2, Sp):
    Bx, Tx, C = x.shape
    xf = x.reshape(-1, C)
    Wg_pad = jnp.pad(Wg, ((0, 0), (0, 128 - E)))
    scores, y = _run(xf, Wg_pad, W1, W2, Wp, S1, S2, Sp)
    return y.reshape(Bx, Tx, C), scores


# speedup vs baseline: 1.0028x; 1.0028x over previous
import jax
import jax.numpy as jnp
from jax.experimental import pallas as pl
from jax.experimental.pallas import tpu as pltpu

D = 1024
F_EXP = 2048
F_SH = 4096
E = 16
N = 32
FBLK = 1024
HF = FBLK // 2
CPE = F_EXP // FBLK
UNITS = E + F_SH // F_EXP


def _moe_kernel(x_ref, wg_ref, w1a_ref, w1b_ref, w2a_ref, w2b_ref,
                wpa_ref, wpb_ref, s1a_ref, s1b_ref, s2a_ref, s2b_ref,
                spa_ref, spb_ref, scores_ref, y_ref, w_scr):
    u = pl.program_id(0)
    f = pl.program_id(1)
    first = jnp.logical_and(u == 0, f == 0)

    @pl.when(first)
    def _gating():
        xf = x_ref[:]
        scores = jnp.dot(xf, wg_ref[:], preferred_element_type=jnp.float32)
        scores_ref[:] = scores[:, :E]
        cols = jax.lax.broadcasted_iota(jnp.int32, (N, 128), 1)
        valid = cols < E
        s_masked = jnp.where(valid, scores, -jnp.inf)
        m = jnp.max(s_masked, axis=1, keepdims=True)
        ex = jnp.where(valid, jnp.exp(s_masked - m), 0.0)
        probs = ex / jnp.sum(ex, axis=1, keepdims=True)
        p1 = jnp.max(jnp.where(valid, probs, -jnp.inf), axis=1, keepdims=True)
        i1 = jnp.min(jnp.where(probs == p1, cols, 128), axis=1, keepdims=True)
        probs2 = jnp.where(cols == i1, -jnp.inf,
                           jnp.where(valid, probs, -jnp.inf))
        p2 = jnp.max(probs2, axis=1, keepdims=True)
        i2 = jnp.min(jnp.where(probs2 == p2, cols, 128), axis=1, keepdims=True)
        sel = jnp.logical_or(cols == i1, cols == i2)
        w = jnp.where(sel, probs, 0.0)
        w = jnp.where(jnp.logical_and(cols >= E, cols < UNITS), 1.0, w)
        w_scr[:] = w
        y_ref[:] = jnp.zeros_like(y_ref)

    xf = x_ref[:]
    cols = jax.lax.broadcasted_iota(jnp.int32, (N, 128), 1)
    wu = jnp.sum(jnp.where(cols == u, w_scr[:], 0.0), axis=1, keepdims=True)

    def ffn(a, b, c):
        xh1 = jnp.dot(xf, a, preferred_element_type=jnp.float32)
        xh2 = jnp.dot(xf, b, preferred_element_type=jnp.float32)
        h = (xh1 * jax.nn.sigmoid(xh1)) * xh2
        y_ref[:] += jnp.dot(h, c, preferred_element_type=jnp.float32) * wu

    @pl.when(u < E)
    def _expert():
        ffn(w1a_ref[0], w2a_ref[0], wpa_ref[0])
        ffn(w1b_ref[0], w2b_ref[0], wpb_ref[0])

    @pl.when(u >= E)
    def _shared():
        ffn(s1a_ref[:], s2a_ref[:], spa_ref[:])
        ffn(s1b_ref[:], s2b_ref[:], spb_ref[:])


@jax.jit
def _run(xf, Wg_pad, W1, W2, Wp, S1, S2, Sp):
    def w_map(half):
        def m(u, f):
            e = jnp.minimum(u, E - 1)
            fc = jnp.where(u < E, f, CPE - 1)
            return (e, 0, 2 * fc + half)
        return m

    def wp_map(half):
        def m(u, f):
            e = jnp.minimum(u, E - 1)
            fc = jnp.where(u < E, f, CPE - 1)
            return (e, 2 * fc + half, 0)
        return m

    def s_map(half):
        def m(u, f):
            j = jnp.where(u < E, 0, (u - E) * CPE + f)
            return (0, 2 * j + half)
        return m

    def sp_map(half):
        def m(u, f):
            j = jnp.where(u < E, 0, (u - E) * CPE + f)
            return (2 * j + half, 0)
        return m

    scores, y = pl.pallas_call(
        _moe_kernel,
        grid=(UNITS, CPE),
        in_specs=[
            pl.BlockSpec((N, D), lambda u, f: (0, 0)),
            pl.BlockSpec((D, 128), lambda u, f: (0, 0)),
            pl.BlockSpec((1, D, HF), w_map(0)),
            pl.BlockSpec((1, D, HF), w_map(1)),
            pl.BlockSpec((1, D, HF), w_map(0)),
            pl.BlockSpec((1, D, HF), w_map(1)),
            pl.BlockSpec((1, HF, D), wp_map(0)),
            pl.BlockSpec((1, HF, D), wp_map(1)),
            pl.BlockSpec((D, HF), s_map(0)),
            pl.BlockSpec((D, HF), s_map(1)),
            pl.BlockSpec((D, HF), s_map(0)),
            pl.BlockSpec((D, HF), s_map(1)),
            pl.BlockSpec((HF, D), sp_map(0)),
            pl.BlockSpec((HF, D), sp_map(1)),
        ],
        out_specs=[
            pl.BlockSpec((N, E), lambda u, f: (0, 0)),
            pl.BlockSpec((N, D), lambda u, f: (0, 0)),
        ],
        out_shape=[
            jax.ShapeDtypeStruct((N, E), jnp.float32),
            jax.ShapeDtypeStruct((N, D), jnp.float32),
        ],
        scratch_shapes=[pltpu.VMEM((N, 128), jnp.float32)],
        compiler_params=pltpu.CompilerParams(
            dimension_semantics=("arbitrary", "arbitrary"),
        ),
    )(xf, Wg_pad, W1, W1, W2, W2, Wp, Wp, S1, S1, S2, S2, Sp, Sp)
    return scores, y


def kernel(x, Wg, W1, W2, Wp, S1, S2, Sp):
    Bx, Tx, C = x.shape
    xf = x.reshape(-1, C)
    Wg_pad = jnp.pad(Wg, ((0, 0), (0, 128 - E)))
    scores, y = _run(xf, Wg_pad, W1, W2, Wp, S1, S2, Sp)
    return y.reshape(Bx, Tx, C), scores
